# Initial kernel scaffold; baseline (speedup 1.0000x reference)
import jax
import jax.numpy as jnp
from jax.experimental import pallas as pl


def kernel(x, causal_view, diffusion_view, edge_index, gat1_W, gat1_asrc, gat1_adst, gat1_b, gat2_W, gat2_asrc, gat2_adst, gat2_b, origin_W1, origin_b1, origin_W2, origin_b2, causal_W1, causal_b1, causal_W2, causal_b2, diff_W1, diff_b1, diff_W2, diff_b2, adj_W1, adj_b1, adj_W2, adj_b2, adj_W3, adj_b3):
    raise NotImplementedError("write your pallas kernel here")


# speedup vs baseline: 15.9517x; 15.9517x over previous
import functools

import jax
import jax.numpy as jnp
from jax import lax
from jax.experimental import pallas as pl
from jax.experimental.pallas import tpu as pltpu
from jax.experimental.pallas import tpu_sc as plsc

NN = 10000
EE = 320000
DIN = 128
HID = 128
DOUT = 64
NHEADS = 4
HCH = 32
TTEMP = 0.07

NP = 10240
SC_NC = 2
SC_NS = 16
CHUNK = 128
NCHUNK = EE // CHUNK
ROWS_PER_SUB = NP // SC_NS


def _leaky(e):
    return jnp.where(e > 0, e, 0.2 * e)


def _elu(x):
    return jnp.where(x > 0, x, jnp.exp(x) - 1.0)


def _k1_body(x_ref, w_ref, asrc_ref, adst_ref,
             h1_ref, st_ref, stab_ref, zl_ref, ol_ref):
    xv = x_ref[0]
    h1 = jnp.dot(xv, w_ref[...], preferred_element_type=jnp.float32)
    h1_ref[0, 0:NN, :] = h1
    asrc = asrc_ref[...]
    adst = adst_ref[...]
    stabs = []
    for h in range(NHEADS):
        hc = h1[:, h * HCH:(h + 1) * HCH]
        ss = jnp.sum(hc * asrc[h, :][None, :], axis=1)
        sd = jnp.sum(hc * adst[h, :][None, :], axis=1)
        st_ref[0, 0:NN, h:h + 1] = ss[:, None]
        st_ref[0, 0:NN, NHEADS + h:NHEADS + h + 1] = sd[:, None]
        m = jnp.maximum(jnp.max(ss) + jnp.max(sd), 0.0)
        stabs.append(m)
        el = _leaky(ss + sd)
        p = jnp.exp(el - m)
        zl_ref[0, :, h:h + 1] = p[:, None]
        ol_ref[0, :, h * HCH:(h + 1) * HCH] = hc * p[:, None]
    stab_ref[0, 0, :] = jnp.stack(stabs)


def _k1_call(views, w1, asrc, adst):
    return pl.pallas_call(
        _k1_body,
        grid=(3,),
        compiler_params=pltpu.CompilerParams(vmem_limit_bytes=110 * 2**20),
        in_specs=[
            pl.BlockSpec((1, NN, DIN), lambda v: (v, 0, 0)),
            pl.BlockSpec((DIN, HID), lambda v: (0, 0)),
            pl.BlockSpec((NHEADS, HCH), lambda v: (0, 0)),
            pl.BlockSpec((NHEADS, HCH), lambda v: (0, 0)),
        ],
        out_specs=[
            pl.BlockSpec((1, NP, HID), lambda v: (v, 0, 0)),
            pl.BlockSpec((1, NP, 16), lambda v: (v, 0, 0)),
            pl.BlockSpec((1, 1, NHEADS), lambda v: (v, 0, 0)),
            pl.BlockSpec((1, NN, NHEADS), lambda v: (v, 0, 0)),
            pl.BlockSpec((1, NN, HID), lambda v: (v, 0, 0)),
        ],
        out_shape=[
            jax.ShapeDtypeStruct((3, NP, HID), jnp.float32),
            jax.ShapeDtypeStruct((3, NP, 16), jnp.float32),
            jax.ShapeDtypeStruct((3, 1, NHEADS), jnp.float32),
            jax.ShapeDtypeStruct((3, NN, NHEADS), jnp.float32),
            jax.ShapeDtypeStruct((3, NN, HID), jnp.float32),
        ],
    )(views, w1, asrc, adst)


def _make_edge_pass(D, heads):
    tabcols = 2 * heads
    cph = D // heads
    mesh = plsc.VectorSubcoreMesh(core_axis_name="c", subcore_axis_name="s")

    @functools.partial(
        pl.kernel, mesh=mesh,
        compiler_params=pltpu.CompilerParams(
            needs_layout_passes=False, use_tc_tiling_on_sc=False),
        out_type=[
            jax.ShapeDtypeStruct((SC_NC, NP, D), jnp.float32),
            jax.ShapeDtypeStruct((SC_NC, NP, 16), jnp.float32),
        ],
        scratch_types=[
            pltpu.VMEM((CHUNK,), jnp.int32),
            pltpu.VMEM((CHUNK,), jnp.int32),
            pltpu.VMEM((CHUNK, D), jnp.float32),
            pltpu.VMEM((CHUNK, 16), jnp.float32),
            pltpu.VMEM((CHUNK, 16), jnp.float32),
            pltpu.VMEM((CHUNK, 16), jnp.float32),
            pltpu.VMEM((heads, 16), jnp.float32),
            pltpu.VMEM_SHARED((NP, D), jnp.float32),
            pltpu.VMEM_SHARED((NP, 16), jnp.float32),
        ],
    )
    def edge_pass(src_hbm, dst_hbm, h_hbm, st_hbm, stab_hbm, zd_hbm, z16_hbm,
                  out_hbm, outz_hbm,
                  src_v, dst_v, rows_v, pz_v, sts_v, std_v, stab_v, acc_o, acc_z):
        core = lax.axis_index("c")
        sub = lax.axis_index("s")
        wid = sub * SC_NC + core
        base = sub * ROWS_PER_SUB
        pltpu.sync_copy(zd_hbm.at[pl.ds(base, ROWS_PER_SUB)],
                        acc_o.at[pl.ds(base, ROWS_PER_SUB)])
        pltpu.sync_copy(z16_hbm.at[pl.ds(base, ROWS_PER_SUB)],
                        acc_z.at[pl.ds(base, ROWS_PER_SUB)])
        pltpu.sync_copy(stab_hbm, stab_v)
        iota = lax.iota(jnp.int32, 16)
        zero16 = jnp.zeros((16,), jnp.float32)

        def zb(i, c):
            pz_v[i, :] = zero16
            return c

        lax.fori_loop(0, CHUNK, zb, 0)
        plsc.subcore_barrier()

        trips = (NCHUNK - wid + 31) // 32

        def chunk_body(t, carry):
            cidx = wid + t * 32
            off = cidx * CHUNK
            pltpu.sync_copy(src_hbm.at[pl.ds(off, CHUNK)], src_v)
            pltpu.sync_copy(dst_hbm.at[pl.ds(off, CHUNK)], dst_v)
            pltpu.sync_copy(h_hbm.at[src_v], rows_v)
            pltpu.sync_copy(st_hbm.at[src_v], sts_v)
            pltpu.sync_copy(st_hbm.at[dst_v], std_v)

            def group_body(g, gc):
                rowidx = g * 16 + iota
                for h in range(heads):
                    hcol = jnp.full((16,), h, jnp.int32)
                    ss = plsc.load_gather(sts_v, [rowidx, hcol])
                    sd = plsc.load_gather(std_v, [rowidx, hcol + heads])
                    e = _leaky(ss + sd)
                    p = jnp.exp(e - stab_v[h, :])
                    plsc.store_scatter(pz_v, [rowidx, hcol], p)
                    for j in range(h * cph, (h + 1) * cph):
                        jcol = jnp.full((16,), j, jnp.int32)
                        rv = plsc.load_gather(rows_v, [rowidx, jcol])
                        plsc.store_scatter(rows_v, [rowidx, jcol], rv * p)
                return gc

            lax.fori_loop(0, CHUNK // 16, group_body, 0)
            pltpu.sync_copy(rows_v, acc_o.at[dst_v], add=True)
            pltpu.sync_copy(pz_v, acc_z.at[dst_v], add=True)
            return carry

        lax.fori_loop(0, trips, chunk_body, 0)
        plsc.subcore_barrier()
        pltpu.sync_copy(acc_o.at[pl.ds(base, ROWS_PER_SUB)],
                        out_hbm.at[core, pl.ds(base, ROWS_PER_SUB)])
        pltpu.sync_copy(acc_z.at[pl.ds(base, ROWS_PER_SUB)],
                        outz_hbm.at[core, pl.ds(base, ROWS_PER_SUB)])

    return edge_pass


_edge_pass_l1 = _make_edge_pass(HID, NHEADS)
_edge_pass_l2 = _make_edge_pass(DOUT, 1)


_K3_RB = 2000
_K3_NB = NN // _K3_RB


def _k3a_body(op_ref, oz_ref, ol_ref, zl_ref, b1_ref, w2_ref, a2s_ref, a2d_ref,
              h2_ref, st2_ref, bm_ref):
    g1 = op_ref[0] + op_ref[1] + ol_ref[...]
    z = oz_ref[0, :, 0:NHEADS] + oz_ref[1, :, 0:NHEADS] + zl_ref[...]
    b1 = b1_ref[...]
    chunks = []
    for h in range(NHEADS):
        gh = g1[:, h * HCH:(h + 1) * HCH] / (z[:, h:h + 1] + 1e-16)
        chunks.append(_elu(gh + b1[0, h * HCH:(h + 1) * HCH][None, :]))
    x2 = jnp.concatenate(chunks, axis=1)
    h2 = jnp.dot(x2, w2_ref[...], preferred_element_type=jnp.float32)
    h2_ref[...] = h2
    ss = jnp.sum(h2 * a2s_ref[...], axis=1)
    sd = jnp.sum(h2 * a2d_ref[...], axis=1)
    st2_ref[:, 0:1] = ss[:, None]
    st2_ref[:, 1:2] = sd[:, None]
    bm_ref[0, 0, :] = jnp.stack(
        [jnp.max(ss), jnp.max(sd), 0.0, 0.0, 0.0, 0.0, 0.0, 0.0])


def _k3a_call(op, oz, ol, zl, b1, w2, a2s, a2d):
    return pl.pallas_call(
        _k3a_body,
        grid=(_K3_NB,),
        in_specs=[
            pl.BlockSpec((2, _K3_RB, HID), lambda i: (0, i, 0)),
            pl.BlockSpec((2, _K3_RB, 16), lambda i: (0, i, 0)),
            pl.BlockSpec((_K3_RB, HID), lambda i: (i, 0)),
            pl.BlockSpec((_K3_RB, NHEADS), lambda i: (i, 0)),
            pl.BlockSpec((1, HID), lambda i: (0, 0)),
            pl.BlockSpec((HID, DOUT), lambda i: (0, 0)),
            pl.BlockSpec((1, DOUT), lambda i: (0, 0)),
            pl.BlockSpec((1, DOUT), lambda i: (0, 0)),
        ],
        out_specs=[
            pl.BlockSpec((_K3_RB, DOUT), lambda i: (i, 0)),
            pl.BlockSpec((_K3_RB, 16), lambda i: (i, 0)),
            pl.BlockSpec((1, 1, 8), lambda i: (i, 0, 0)),
        ],
        out_shape=[
            jax.ShapeDtypeStruct((NP, DOUT), jnp.float32),
            jax.ShapeDtypeStruct((NP, 16), jnp.float32),
            jax.ShapeDtypeStruct((_K3_NB, 1, 8), jnp.float32),
        ],
    )(op, oz, ol, zl, b1, w2, a2s, a2d)


def _k3b_body(h2_ref, st2_ref, bm_ref, stab_ref, zl2_ref, ol2_ref):
    bm = bm_ref[...]
    m = jnp.maximum(jnp.max(bm[:, 0, 0]) + jnp.max(bm[:, 0, 1]), 0.0)
    stab_ref[0, :] = jnp.full((8,), m)
    ss = st2_ref[0:NN, 0:1]
    sd = st2_ref[0:NN, 1:2]
    el = _leaky(ss + sd)
    p = jnp.exp(el - m)
    zl2_ref[...] = p
    ol2_ref[...] = h2_ref[0:NN, :] * p


def _k3b_call(h2, st2, bm):
    return pl.pallas_call(
        _k3b_body,
        out_shape=[
            jax.ShapeDtypeStruct((1, 8), jnp.float32),
            jax.ShapeDtypeStruct((NN, 1), jnp.float32),
            jax.ShapeDtypeStruct((NN, DOUT), jnp.float32),
        ],
    )(h2, st2, bm)


_K5A_RB = 1000
_K5A_NB = NN // _K5A_RB
_K5B_RB = 2000
_K5B_NI = NN // _K5B_RB
_K5B_CB = 2048
_K5B_NJ = NP // _K5B_CB


def _k5a_body(p2x_ref, z2x_ref, ol2x_ref, zl2x_ref,
              p2c_ref, z2c_ref, ol2c_ref, zl2c_ref,
              p2d_ref, z2d_ref, ol2d_ref, zl2d_ref,
              b2_ref,
              ow1_ref, ob1_ref, ow2_ref, ob2_ref,
              cw1_ref, cb1_ref, cw2_ref, cb2_ref,
              dw1_ref, db1_ref, dw2_ref, db2_ref,
              aw1_ref, ab1_ref, aw2_ref, ab2_ref, aw3_ref, ab3_ref,
              v1_ref, v2_ref, part_ref):
    b2 = b2_ref[...]
    encs = []
    for (p2, z2, ol2, zl2, w1r, b1r, w2r, b2r) in (
            (p2x_ref, z2x_ref, ol2x_ref, zl2x_ref, ow1_ref, ob1_ref, ow2_ref, ob2_ref),
            (p2c_ref, z2c_ref, ol2c_ref, zl2c_ref, cw1_ref, cb1_ref, cw2_ref, cb2_ref),
            (p2d_ref, z2d_ref, ol2d_ref, zl2d_ref, dw1_ref, db1_ref, dw2_ref, db2_ref)):
        g2 = p2[0] + p2[1] + ol2[...]
        z = z2[0, :, 0:1] + z2[1, :, 0:1] + zl2[...]
        x3 = _elu(g2 / (z + 1e-16) + b2)
        hmid = jnp.maximum(
            jnp.dot(x3, w1r[...], preferred_element_type=jnp.float32) + b1r[...], 0.0)
        encs.append(jnp.dot(hmid, w2r[...], preferred_element_type=jnp.float32) + b2r[...])
    o, c, d = encs
    comb = jnp.concatenate([o, c, d], axis=1)
    a1 = jnp.maximum(jnp.dot(comb, aw1_ref[...], preferred_element_type=jnp.float32) + ab1_ref[...], 0.0)
    a2 = jnp.maximum(jnp.dot(a1, aw2_ref[...], preferred_element_type=jnp.float32) + ab2_ref[...], 0.0)
    adj = jnp.dot(a2, aw3_ref[...], preferred_element_type=jnp.float32) + ab3_ref[...]

    no = jnp.sqrt(jnp.sum(o * o, axis=1, keepdims=True))
    na = jnp.sqrt(jnp.sum(adj * adj, axis=1, keepdims=True))
    v1 = o / jnp.maximum(no, 1e-12)
    v2 = adj / jnp.maximum(na, 1e-12)
    v1_ref[...] = v1
    v2_ref[...] = v2

    def cos_sum(x, y):
        nx = jnp.maximum(jnp.sqrt(jnp.sum(x * x, axis=1)), 1e-8)
        ny = jnp.maximum(jnp.sqrt(jnp.sum(y * y, axis=1)), 1e-8)
        return jnp.sum(jnp.sum(x * y, axis=1) / (nx * ny))

    bd = jnp.sum((adj - c) ** 2)
    dg = jnp.sum(v1 * v2) / TTEMP
    part_ref[0, 0, :] = jnp.stack(
        [cos_sum(o, c), cos_sum(o, d), cos_sum(c, d), bd, dg, 0.0, 0.0, 0.0])


def _k5a_call(args):
    rb = _K5A_RB
    vspec = lambda w: pl.BlockSpec((2, rb, w), lambda i: (0, i, 0))
    nspec = lambda w: pl.BlockSpec((rb, w), lambda i: (i, 0))
    wspec = lambda r, c: pl.BlockSpec((r, c), lambda i: (0, 0))
    in_specs = []
    for _ in range(3):
        in_specs += [vspec(DOUT), vspec(16), nspec(DOUT), nspec(1)]
    in_specs += [wspec(1, DOUT)]
    for _ in range(3):
        in_specs += [wspec(DOUT, DOUT), wspec(1, DOUT), wspec(DOUT, DOUT), wspec(1, DOUT)]
    in_specs += [wspec(192, 256), wspec(1, 256), wspec(256, 128), wspec(1, 128),
                 wspec(128, DOUT), wspec(1, DOUT)]
    return pl.pallas_call(
        _k5a_body,
        grid=(_K5A_NB,),
        in_specs=in_specs,
        out_specs=[
            pl.BlockSpec((rb, DOUT), lambda i: (i, 0)),
            pl.BlockSpec((rb, DOUT), lambda i: (i, 0)),
            pl.BlockSpec((1, 1, 8), lambda i: (i, 0, 0)),
        ],
        out_shape=[
            jax.ShapeDtypeStruct((NN, DOUT), jnp.float32),
            jax.ShapeDtypeStruct((NP, DOUT), jnp.float32),
            jax.ShapeDtypeStruct((_K5A_NB, 1, 8), jnp.float32),
        ],
    )(*args)


def _k5b_body(v1_ref, v2_ref, out_ref, acc_ref):
    j = pl.program_id(1)

    @pl.when(j == 0)
    def _():
        acc_ref[...] = jnp.zeros((_K5B_RB, 1), jnp.float32)

    sim = lax.dot_general(v1_ref[...], v2_ref[...], (((1,), (1,)), ((), ())),
                          preferred_element_type=jnp.float32) / TTEMP
    ids = lax.broadcasted_iota(jnp.int32, (1, _K5B_CB), 1) + j * _K5B_CB
    sim = jnp.where(ids < NN, sim, -1e30)
    acc_ref[...] = acc_ref[...] + jnp.sum(jnp.exp(sim), axis=1, keepdims=True)
    out_ref[0, 0, :] = jnp.full((8,), jnp.sum(jnp.log(acc_ref[...])))


def _k5b_call(v1, v2):
    return pl.pallas_call(
        _k5b_body,
        grid=(_K5B_NI, _K5B_NJ),
        in_specs=[
            pl.BlockSpec((_K5B_RB, DOUT), lambda i, j: (i, 0)),
            pl.BlockSpec((_K5B_CB, DOUT), lambda i, j: (j, 0)),
        ],
        out_specs=pl.BlockSpec((1, 1, 8), lambda i, j: (i, 0, 0)),
        out_shape=jax.ShapeDtypeStruct((_K5B_NI, 1, 8), jnp.float32),
        scratch_shapes=[pltpu.VMEM((_K5B_RB, 1), jnp.float32)],
    )(v1, v2)


def _k5c_body(part_ref, lse_ref, co_ref, al_ref, bd_ref):
    part = part_ref[...]
    lse = lse_ref[...]
    n = jnp.float32(NN)
    contrast = jnp.sum(lse[:, 0, 0]) / n - jnp.sum(part[:, 0, 4]) / n
    align = (1.0 - (jnp.sum(part[:, 0, 0]) / n + jnp.sum(part[:, 0, 1]) / n
                    + jnp.sum(part[:, 0, 2]) / n) / 3.0) * 0.4
    backdoor = jnp.sum(part[:, 0, 3]) / (n * DOUT) * 0.3
    co_ref[...] = jnp.full((1, 1), contrast)
    al_ref[...] = jnp.full((1, 1), align)
    bd_ref[...] = jnp.full((1, 1), backdoor)


def _k5c_call(part, lse):
    return pl.pallas_call(
        _k5c_body,
        out_shape=[
            jax.ShapeDtypeStruct((1, 1), jnp.float32),
            jax.ShapeDtypeStruct((1, 1), jnp.float32),
            jax.ShapeDtypeStruct((1, 1), jnp.float32),
        ],
    )(part, lse)


def kernel(x, causal_view, diffusion_view, edge_index,
           gat1_W, gat1_asrc, gat1_adst, gat1_b,
           gat2_W, gat2_asrc, gat2_adst, gat2_b,
           origin_W1, origin_b1, origin_W2, origin_b2,
           causal_W1, causal_b1, causal_W2, causal_b2,
           diff_W1, diff_b1, diff_W2, diff_b2,
           adj_W1, adj_b1, adj_W2, adj_b2, adj_W3, adj_b3):
    src = edge_index[0]
    dst = edge_index[1]
    views = jnp.stack([x, causal_view, diffusion_view])
    h1, st1, stab1, zl1, ol1 = _k1_call(
        views, gat1_W, gat1_asrc.reshape(NHEADS, HCH),
        gat1_adst.reshape(NHEADS, HCH))

    zd128 = jnp.zeros((NP, HID), jnp.float32)
    zd64 = jnp.zeros((NP, DOUT), jnp.float32)
    z16 = jnp.zeros((NP, 16), jnp.float32)

    l2_parts = []
    for v in range(3):
        stabrep = jnp.broadcast_to(stab1[v, 0][:, None], (NHEADS, 16))
        op, oz = _edge_pass_l1(src, dst, h1[v], st1[v], stabrep, zd128, z16)
        h2, st2, bm = _k3a_call(
            op[:, 0:NN], oz[:, 0:NN], ol1[v], zl1[v], gat1_b.reshape(1, HID), gat2_W,
            gat2_asrc.reshape(1, DOUT), gat2_adst.reshape(1, DOUT))
        stab2, zl2, ol2 = _k3b_call(h2, st2, bm)
        stabrep2 = jnp.broadcast_to(stab2[0, 0][None, None], (1, 16))
        op2, oz2 = _edge_pass_l2(src, dst, h2, st2, stabrep2, zd64, z16)
        l2_parts.append((op2[:, 0:NN], oz2[:, 0:NN], ol2, zl2))

    args = []
    for v in range(3):
        args.extend(l2_parts[v])
    args.append(gat2_b.reshape(1, DOUT))
    args.extend([origin_W1, origin_b1.reshape(1, DOUT), origin_W2, origin_b2.reshape(1, DOUT)])
    args.extend([causal_W1, causal_b1.reshape(1, DOUT), causal_W2, causal_b2.reshape(1, DOUT)])
    args.extend([diff_W1, diff_b1.reshape(1, DOUT), diff_W2, diff_b2.reshape(1, DOUT)])
    args.extend([adj_W1, adj_b1.reshape(1, 256), adj_W2, adj_b2.reshape(1, 128),
                 adj_W3, adj_b3.reshape(1, DOUT)])
    v1, v2, part = _k5a_call(args)
    lse = _k5b_call(v1, v2)
    co, al, bd = _k5c_call(part, lse)
    return (co[0, 0], al[0, 0], bd[0, 0])
